# Initial kernel scaffold; baseline (speedup 1.0000x reference)
import jax
import jax.numpy as jnp
from jax.experimental import pallas as pl


def kernel(x, W1, b1, W2, b2, edge_index):
    raise NotImplementedError("write your pallas kernel here")


# speedup vs baseline: 2.3478x; 2.3478x over previous
import functools

import jax
import jax.numpy as jnp
from jax import lax
from jax.experimental import pallas as pl
from jax.experimental.pallas import tpu as pltpu
from jax.experimental.pallas import tpu_sc as plsc

L = 16
NC = 2
NS = 16
NW = NC * NS


def _make_degree_kernel(n_own: int, n_edges: int):
  ept = n_edges // NW
  n_full = ept // L
  rem = ept - n_full * L
  stage = ept + (L - rem) % L

  mesh = plsc.VectorSubcoreMesh(core_axis_name="c", subcore_axis_name="s")

  @functools.partial(
      pl.kernel,
      out_type=jax.ShapeDtypeStruct((NW, n_own), jnp.float32),
      mesh=mesh,
      scratch_types=[
          pltpu.VMEM((stage,), jnp.int32),
          pltpu.VMEM((n_own,), jnp.float32),
      ],
      compiler_params=pltpu.CompilerParams(needs_layout_passes=False, use_tc_tiling_on_sc=False),
  )
  def degree_kernel(dst_hbm, out_hbm, dstv, acc):
    cid = lax.axis_index("c")
    sid = lax.axis_index("s")
    wid = sid * NC + cid
    base = wid * ept

    pltpu.sync_copy(dst_hbm.at[pl.ds(base, ept)], dstv.at[pl.ds(0, ept)])

    zeros = jnp.zeros((L,), jnp.float32)
    def zero_body(i, _):
      acc[pl.ds(i * L, L)] = zeros
      return 0
    lax.fori_loop(0, n_own // L, zero_body, 0)

    def count_group(idx, lane_mask):
      cnt, last = plsc.scan_count(idx, lane_mask)
      m = last if lane_mask is None else (last & lane_mask)
      plsc.addupdate_scatter(acc, [idx], cnt.astype(jnp.float32), mask=m)

    def scan_body(i, _):
      count_group(dstv[pl.ds(i * L, L)], None)
      return 0
    lax.fori_loop(0, n_full, scan_body, 0)
    if rem:
      lanes = lax.iota(jnp.int32, L)
      count_group(dstv[pl.ds(n_full * L, L)], lanes < rem)

    pltpu.sync_copy(acc, out_hbm.at[wid])

  return degree_kernel


def _make_degree_reduce_kernel(n_own: int, ro: int):
  mesh = plsc.VectorSubcoreMesh(core_axis_name="c", subcore_axis_name="s")

  @functools.partial(
      pl.kernel,
      out_type=jax.ShapeDtypeStruct((n_own,), jnp.float32),
      mesh=mesh,
      scratch_types=[
          pltpu.VMEM((ro,), jnp.float32),
          pltpu.VMEM((ro,), jnp.float32),
      ],
      compiler_params=pltpu.CompilerParams(needs_layout_passes=False, use_tc_tiling_on_sc=False),
  )
  def degree_reduce_kernel(partial_hbm, out_hbm, acc, tmp):
    cid = lax.axis_index("c")
    sid = lax.axis_index("s")
    wid = sid * NC + cid
    lo = wid * ro

    zeros = jnp.zeros((L,), jnp.float32)
    for g in range(ro // L):
      acc[pl.ds(g * L, L)] = zeros
    for t in range(NW):
      pltpu.sync_copy(partial_hbm.at[t, pl.ds(lo, ro)], tmp)
      for g in range(ro // L):
        plsc.addupdate(acc.at[pl.ds(g * L, L)], tmp[pl.ds(g * L, L)])
    pltpu.sync_copy(acc, out_hbm.at[pl.ds(lo, ro)])

  return degree_reduce_kernel


def _make_agg_kernel(n_nodes: int, e_pad: int, d: int, ro: int, eb: int,
                     gb: int):
  assert e_pad % eb == 0 and eb % L == 0 and d % L == 0 and gb % L == 0
  n_blocks = e_pad // eb
  n_grp = eb // L
  n_pad = ro * NW

  mesh = plsc.VectorSubcoreMesh(core_axis_name="c", subcore_axis_name="s")

  @functools.partial(
      pl.kernel,
      out_type=jax.ShapeDtypeStruct((n_pad, d), jnp.float32),
      mesh=mesh,
      scratch_types=[
          pltpu.VMEM((eb,), jnp.int32),
          pltpu.VMEM((eb,), jnp.int32),
          pltpu.VMEM((eb + 2 * gb,), jnp.int32),
          pltpu.VMEM((eb + 2 * gb,), jnp.int32),
          pltpu.VMEM((gb, d), jnp.float32),
          pltpu.VMEM((ro + 1, d), jnp.float32),
          pltpu.SemaphoreType.DMA,
      ],
      compiler_params=pltpu.CompilerParams(needs_layout_passes=False, use_tc_tiling_on_sc=False),
  )
  def agg_kernel(g_hbm, src_hbm, dst_hbm, out_hbm,
                 srcb, dstb, srcf, dstf, rows, acc, sem):
    cid = lax.axis_index("c")
    sid = lax.axis_index("s")
    wid = sid * NC + cid
    lo = wid * ro

    zeros = jnp.zeros((L,), jnp.float32)
    def zbody(i, _):
      acc[i // (d // L), pl.ds((i % (d // L)) * L, L)] = zeros
      return 0
    lax.fori_loop(0, (ro + 1) * (d // L), zbody, 0)

    pad_src = jnp.zeros((L,), jnp.int32)
    pad_dst = jnp.full((L,), ro, jnp.int32)

    def block_body(bk, _):
      pltpu.sync_copy(src_hbm.at[pl.ds(bk * eb, eb)], srcb)
      pltpu.sync_copy(dst_hbm.at[pl.ds(bk * eb, eb)], dstb)

      def scan_body(i, off):
        d16 = dstb[pl.ds(i * L, L)]
        s16 = srcb[pl.ds(i * L, L)]
        m = (d16 >= lo) & (d16 < lo + ro)
        plsc.store_compressed(srcf.at[pl.ds(off, L)], s16, mask=m)
        plsc.store_compressed(dstf.at[pl.ds(off, L)], d16 - lo, mask=m)
        return off + jnp.sum(m.astype(jnp.int32))
      off = lax.fori_loop(0, n_grp, scan_body, jnp.int32(0))

      for t in range(gb // L + 1):
        srcf[pl.ds(off + t * L, L)] = pad_src
        dstf[pl.ds(off + t * L, L)] = pad_dst
      nb = (off + gb - 1) // gb

      def batch_body(j, _):
        pltpu.async_copy(g_hbm.at[srcf.at[pl.ds(j * gb, gb)]], rows,
                         sem).wait()
        for t in range(gb // L):
          dv = dstf[pl.ds(j * gb + t * L, L)]
          for l in range(L):
            r = dv[l]
            for k in range(d // L):
              plsc.addupdate(acc.at[r, pl.ds(k * L, L)],
                             rows[t * L + l, pl.ds(k * L, L)])
        return 0
      lax.fori_loop(0, nb, batch_body, 0)
      return 0

    lax.fori_loop(0, n_blocks, block_body, 0)

    pltpu.sync_copy(acc.at[pl.ds(0, ro)], out_hbm.at[pl.ds(lo, ro)])

  return agg_kernel


def _tc_first(deg_ref, x_ref, w_ref, g_ref, dinv_ref):
  dinv = lax.rsqrt(1.0 + deg_ref[...])
  dinv_ref[...] = dinv
  g_ref[...] = jnp.dot(x_ref[...] * dinv, w_ref[...],
                       preferred_element_type=jnp.float32)


def _tc_mid(agg1_ref, agg2_ref, g_ref, dinv_ref, b_ref, w_ref, out_ref):
  t = dinv_ref[...]
  agg = jnp.concatenate([agg1_ref[...], agg2_ref[...]], axis=1)
  u = jnp.maximum(t * (agg + g_ref[...]) + b_ref[...], 0.0)
  out_ref[...] = jnp.dot(u * t, w_ref[...],
                         preferred_element_type=jnp.float32)


def _tc_last(agg_ref, g_ref, dinv_ref, b_ref, out_ref):
  out_ref[...] = dinv_ref[...] * (agg_ref[...] + g_ref[...]) + b_ref[...]


def _row_spec(bm, cols):
  return pl.BlockSpec((bm, cols), lambda i: (i, 0))


def _full_spec(r, c):
  return pl.BlockSpec((r, c), lambda i: (0, 0))


def kernel(x, W1, b1, W2, b2, edge_index):
  n, d_in = x.shape
  d_hid = W1.shape[1]
  d_out = W2.shape[1]
  e = edge_index.shape[1]
  src = edge_index[0]
  dst = edge_index[1]

  bm = 1000
  grid = (n // bm,)

  ro = 320
  n_own = ro * NW
  degp = _make_degree_kernel(n_own, e)(dst)
  deg = _make_degree_reduce_kernel(n_own, ro)(degp)[:n, None]

  g1, dinv = pl.pallas_call(
      _tc_first,
      grid=grid,
      in_specs=[_row_spec(bm, 1),
                _row_spec(bm, d_in), _full_spec(d_in, d_hid)],
      out_specs=[_row_spec(bm, d_hid), _row_spec(bm, 1)],
      out_shape=[jax.ShapeDtypeStruct((n, d_hid), jnp.float32),
                 jax.ShapeDtypeStruct((n, 1), jnp.float32)],
  )(deg, x, W1)

  eb, gb = 4096, 32
  e_pad = ((e + eb - 1) // eb) * eb
  src_p = jnp.concatenate([src, jnp.zeros((e_pad - e,), jnp.int32)])
  dst_p = jnp.concatenate([dst, jnp.full((e_pad - e,), n_own, jnp.int32)])

  dh = d_hid // 2
  agg_h = _make_agg_kernel(n, e_pad, dh, ro=ro, eb=eb, gb=gb)
  agg1a = agg_h(g1[:, :dh], src_p, dst_p)[:n]
  agg1b = agg_h(g1[:, dh:], src_p, dst_p)[:n]

  g2 = pl.pallas_call(
      _tc_mid,
      grid=grid,
      in_specs=[_row_spec(bm, dh), _row_spec(bm, dh), _row_spec(bm, d_hid),
                _row_spec(bm, 1), _full_spec(1, d_hid),
                _full_spec(d_hid, d_out)],
      out_specs=_row_spec(bm, d_out),
      out_shape=jax.ShapeDtypeStruct((n, d_out), jnp.float32),
  )(agg1a, agg1b, g1, dinv, b1[None, :], W2)

  agg2 = _make_agg_kernel(n, e_pad, d_out, ro=ro, eb=eb, gb=gb)(
      g2, src_p, dst_p)[:n]

  out = pl.pallas_call(
      _tc_last,
      grid=grid,
      in_specs=[_row_spec(bm, d_out), _row_spec(bm, d_out), _row_spec(bm, 1),
                _full_spec(1, d_out)],
      out_specs=_row_spec(bm, d_out),
      out_shape=jax.ShapeDtypeStruct((n, d_out), jnp.float32),
  )(agg2, g2, dinv, b2[None, :])

  return out


# speedup vs baseline: 2.4280x; 1.0342x over previous
import functools

import jax
import jax.numpy as jnp
from jax import lax
from jax.experimental import pallas as pl
from jax.experimental.pallas import tpu as pltpu
from jax.experimental.pallas import tpu_sc as plsc

L = 16
NC = 2
NS = 16
NW = NC * NS


def _make_degree_kernel(n_own: int, n_edges: int):
  ept = n_edges // NW
  n_full = ept // L
  rem = ept - n_full * L
  stage = ept + (L - rem) % L

  mesh = plsc.VectorSubcoreMesh(core_axis_name="c", subcore_axis_name="s")

  @functools.partial(
      pl.kernel,
      out_type=jax.ShapeDtypeStruct((NW, n_own), jnp.float32),
      mesh=mesh,
      scratch_types=[
          pltpu.VMEM((stage,), jnp.int32),
          pltpu.VMEM((n_own,), jnp.float32),
      ],
      compiler_params=pltpu.CompilerParams(needs_layout_passes=False, use_tc_tiling_on_sc=False),
  )
  def degree_kernel(dst_hbm, out_hbm, dstv, acc):
    cid = lax.axis_index("c")
    sid = lax.axis_index("s")
    wid = sid * NC + cid
    base = wid * ept

    pltpu.sync_copy(dst_hbm.at[pl.ds(base, ept)], dstv.at[pl.ds(0, ept)])

    zeros = jnp.zeros((L,), jnp.float32)
    def zero_body(i, _):
      acc[pl.ds(i * L, L)] = zeros
      return 0
    lax.fori_loop(0, n_own // L, zero_body, 0)

    def count_group(idx, lane_mask):
      cnt, last = plsc.scan_count(idx, lane_mask)
      m = last if lane_mask is None else (last & lane_mask)
      plsc.addupdate_scatter(acc, [idx], cnt.astype(jnp.float32), mask=m)

    def scan_body(i, _):
      count_group(dstv[pl.ds(i * L, L)], None)
      return 0
    lax.fori_loop(0, n_full, scan_body, 0)
    if rem:
      lanes = lax.iota(jnp.int32, L)
      count_group(dstv[pl.ds(n_full * L, L)], lanes < rem)

    pltpu.sync_copy(acc, out_hbm.at[wid])

  return degree_kernel


def _make_degree_reduce_kernel(n_own: int, ro: int):
  mesh = plsc.VectorSubcoreMesh(core_axis_name="c", subcore_axis_name="s")

  @functools.partial(
      pl.kernel,
      out_type=jax.ShapeDtypeStruct((n_own,), jnp.float32),
      mesh=mesh,
      scratch_types=[
          pltpu.VMEM((ro,), jnp.float32),
          pltpu.VMEM((ro,), jnp.float32),
      ],
      compiler_params=pltpu.CompilerParams(needs_layout_passes=False, use_tc_tiling_on_sc=False),
  )
  def degree_reduce_kernel(partial_hbm, out_hbm, acc, tmp):
    cid = lax.axis_index("c")
    sid = lax.axis_index("s")
    wid = sid * NC + cid
    lo = wid * ro

    zeros = jnp.zeros((L,), jnp.float32)
    for g in range(ro // L):
      acc[pl.ds(g * L, L)] = zeros
    for t in range(NW):
      pltpu.sync_copy(partial_hbm.at[t, pl.ds(lo, ro)], tmp)
      for g in range(ro // L):
        plsc.addupdate(acc.at[pl.ds(g * L, L)], tmp[pl.ds(g * L, L)])
    pltpu.sync_copy(acc, out_hbm.at[pl.ds(lo, ro)])

  return degree_reduce_kernel


def _make_agg_kernel(n_nodes: int, e_pad: int, d: int, ro: int, eb: int,
                     gb: int):
  assert e_pad % eb == 0 and eb % L == 0 and d % L == 0 and gb % L == 0
  n_blocks = e_pad // eb
  assert n_blocks % 2 == 0
  n_grp = eb // L
  n_pad = ro * NW
  dl = d // L

  mesh = plsc.VectorSubcoreMesh(core_axis_name="c", subcore_axis_name="s")

  @functools.partial(
      pl.kernel,
      out_type=jax.ShapeDtypeStruct((n_pad, d), jnp.float32),
      mesh=mesh,
      scratch_types=[
          pltpu.VMEM((2, eb), jnp.int32),
          pltpu.VMEM((2, eb), jnp.int32),
          pltpu.VMEM((eb + 2 * gb,), jnp.int32),
          pltpu.VMEM((eb + 2 * gb,), jnp.int32),
          pltpu.VMEM((2, gb, d), jnp.float32),
          pltpu.VMEM((ro + 1, d), jnp.float32),
          pltpu.SemaphoreType.DMA,
          pltpu.SemaphoreType.DMA,
      ],
      compiler_params=pltpu.CompilerParams(needs_layout_passes=False, use_tc_tiling_on_sc=False),
  )
  def agg_kernel(g_hbm, src_hbm, dst_hbm, out_hbm,
                 srcb, dstb, srcf, dstf, rows, acc, sem_e, sem_g):
    cid = lax.axis_index("c")
    sid = lax.axis_index("s")
    wid = sid * NC + cid
    lo = wid * ro

    zeros = jnp.zeros((L,), jnp.float32)
    def zbody(i, _):
      acc[i // dl, pl.ds((i % dl) * L, L)] = zeros
      return 0
    lax.fori_loop(0, (ro + 1) * dl, zbody, 0)

    pad_src = jnp.zeros((L,), jnp.int32)
    pad_dst = jnp.full((L,), ro, jnp.int32)

    def start_edges(bk, par):
      pltpu.async_copy(src_hbm.at[pl.ds(bk * eb, eb)], srcb.at[par], sem_e)
      pltpu.async_copy(dst_hbm.at[pl.ds(bk * eb, eb)], dstb.at[par], sem_e)

    def wait_edges(par):
      pltpu.make_async_copy(src_hbm.at[pl.ds(0, eb)], srcb.at[par],
                            sem_e).wait()
      pltpu.make_async_copy(dst_hbm.at[pl.ds(0, eb)], dstb.at[par],
                            sem_e).wait()

    def start_gather(j, par):
      pltpu.async_copy(g_hbm.at[srcf.at[pl.ds(j * gb, gb)]], rows.at[par],
                       sem_g)

    def wait_gather(par):
      pltpu.make_async_copy(g_hbm.at[srcf.at[pl.ds(0, gb)]], rows.at[par],
                            sem_g).wait()

    def accum(par, j):
      for t in range(gb // L):
        dv = dstf[pl.ds(j * gb + t * L, L)]
        for l in range(L):
          r = dv[l]
          for k in range(dl):
            plsc.addupdate(acc.at[r, pl.ds(k * L, L)],
                           rows[par, t * L + l, pl.ds(k * L, L)])

    def process_block(bk, par):
      wait_edges(par)
      @pl.when(bk + 1 < n_blocks)
      def _():
        start_edges(bk + 1, 1 - par)

      def scan_body(i, off):
        d16 = dstb[par, pl.ds(i * L, L)]
        s16 = srcb[par, pl.ds(i * L, L)]
        doff = d16 - lo
        m = doff.astype(jnp.uint32) < jnp.uint32(ro)
        plsc.store_compressed(srcf.at[pl.ds(off, L)], s16, mask=m)
        plsc.store_compressed(dstf.at[pl.ds(off, L)], doff, mask=m)
        return off + plsc.all_reduce_population_count(m)[0]
      off = lax.fori_loop(0, n_grp, scan_body, jnp.int32(0))

      for t in range(gb // L + 1):
        srcf[pl.ds(off + t * L, L)] = pad_src
        dstf[pl.ds(off + t * L, L)] = pad_dst
      nb = (off + gb - 1) // gb

      @pl.when(nb > 0)
      def _():
        start_gather(0, 0)
      def pair_body(j2, _):
        j = 2 * j2
        wait_gather(0)
        @pl.when(j + 1 < nb)
        def _():
          start_gather(j + 1, 1)
        accum(0, j)
        @pl.when(j + 1 < nb)
        def _():
          wait_gather(1)
          @pl.when(j + 2 < nb)
          def _():
            start_gather(j + 2, 0)
          accum(1, j + 1)
        return 0
      lax.fori_loop(0, (nb + 1) // 2, pair_body, 0)

    start_edges(0, 0)
    def block_pair(b2, _):
      process_block(2 * b2, 0)
      process_block(2 * b2 + 1, 1)
      return 0
    lax.fori_loop(0, n_blocks // 2, block_pair, 0)

    pltpu.sync_copy(acc.at[pl.ds(0, ro)], out_hbm.at[pl.ds(lo, ro)])

  return agg_kernel


def _tc_first(deg_ref, x_ref, w_ref, g_ref, dinv_ref):
  dinv = lax.rsqrt(1.0 + deg_ref[...])
  dinv_ref[...] = dinv
  g_ref[...] = jnp.dot(x_ref[...] * dinv, w_ref[...],
                       preferred_element_type=jnp.float32)


def _tc_mid(agg1_ref, agg2_ref, g_ref, dinv_ref, b_ref, w_ref, out_ref):
  t = dinv_ref[...]
  agg = jnp.concatenate([agg1_ref[...], agg2_ref[...]], axis=1)
  u = jnp.maximum(t * (agg + g_ref[...]) + b_ref[...], 0.0)
  out_ref[...] = jnp.dot(u * t, w_ref[...],
                         preferred_element_type=jnp.float32)


def _tc_last(agg_ref, g_ref, dinv_ref, b_ref, out_ref):
  out_ref[...] = dinv_ref[...] * (agg_ref[...] + g_ref[...]) + b_ref[...]


def _row_spec(bm, cols):
  return pl.BlockSpec((bm, cols), lambda i: (i, 0))


def _full_spec(r, c):
  return pl.BlockSpec((r, c), lambda i: (0, 0))


def kernel(x, W1, b1, W2, b2, edge_index):
  n, d_in = x.shape
  d_hid = W1.shape[1]
  d_out = W2.shape[1]
  e = edge_index.shape[1]
  src = edge_index[0]
  dst = edge_index[1]

  bm = 1000
  grid = (n // bm,)

  ro = 320
  n_own = ro * NW
  degp = _make_degree_kernel(n_own, e)(dst)
  deg = _make_degree_reduce_kernel(n_own, ro)(degp)[:n, None]

  g1, dinv = pl.pallas_call(
      _tc_first,
      grid=grid,
      in_specs=[_row_spec(bm, 1),
                _row_spec(bm, d_in), _full_spec(d_in, d_hid)],
      out_specs=[_row_spec(bm, d_hid), _row_spec(bm, 1)],
      out_shape=[jax.ShapeDtypeStruct((n, d_hid), jnp.float32),
                 jax.ShapeDtypeStruct((n, 1), jnp.float32)],
  )(deg, x, W1)

  eb, gb = 4096, 32
  e_pad = ((e + eb - 1) // eb) * eb
  src_p = jnp.concatenate([src, jnp.zeros((e_pad - e,), jnp.int32)])
  dst_p = jnp.concatenate([dst, jnp.full((e_pad - e,), n_own, jnp.int32)])

  dh = d_hid // 2
  agg_h = _make_agg_kernel(n, e_pad, dh, ro=ro, eb=eb, gb=gb)
  agg1a = agg_h(g1[:, :dh], src_p, dst_p)[:n]
  agg1b = agg_h(g1[:, dh:], src_p, dst_p)[:n]

  g2 = pl.pallas_call(
      _tc_mid,
      grid=grid,
      in_specs=[_row_spec(bm, dh), _row_spec(bm, dh), _row_spec(bm, d_hid),
                _row_spec(bm, 1), _full_spec(1, d_hid),
                _full_spec(d_hid, d_out)],
      out_specs=_row_spec(bm, d_out),
      out_shape=jax.ShapeDtypeStruct((n, d_out), jnp.float32),
  )(agg1a, agg1b, g1, dinv, b1[None, :], W2)

  agg2 = _make_agg_kernel(n, e_pad, d_out, ro=ro, eb=eb, gb=gb)(
      g2, src_p, dst_p)[:n]

  out = pl.pallas_call(
      _tc_last,
      grid=grid,
      in_specs=[_row_spec(bm, d_out), _row_spec(bm, d_out), _row_spec(bm, 1),
                _full_spec(1, d_out)],
      out_specs=_row_spec(bm, d_out),
      out_shape=jax.ShapeDtypeStruct((n, d_out), jnp.float32),
  )(agg2, g2, dinv, b2[None, :])

  return out


# speedup vs baseline: 2.4511x; 1.0095x over previous
import functools

import jax
import jax.numpy as jnp
from jax import lax
from jax.experimental import pallas as pl
from jax.experimental.pallas import tpu as pltpu
from jax.experimental.pallas import tpu_sc as plsc

L = 16
NC = 2
NS = 16
NW = NC * NS


def _make_degree_kernel(n_own: int, n_edges: int):
  ept = n_edges // NW
  n_full = ept // L
  rem = ept - n_full * L
  stage = ept + (L - rem) % L

  mesh = plsc.VectorSubcoreMesh(core_axis_name="c", subcore_axis_name="s")

  @functools.partial(
      pl.kernel,
      out_type=jax.ShapeDtypeStruct((NW, n_own), jnp.float32),
      mesh=mesh,
      scratch_types=[
          pltpu.VMEM((stage,), jnp.int32),
          pltpu.VMEM((n_own,), jnp.float32),
      ],
      compiler_params=pltpu.CompilerParams(needs_layout_passes=False, use_tc_tiling_on_sc=False),
  )
  def degree_kernel(dst_hbm, out_hbm, dstv, acc):
    cid = lax.axis_index("c")
    sid = lax.axis_index("s")
    wid = sid * NC + cid
    base = wid * ept

    pltpu.sync_copy(dst_hbm.at[pl.ds(base, ept)], dstv.at[pl.ds(0, ept)])

    zeros = jnp.zeros((L,), jnp.float32)
    def zero_body(i, _):
      acc[pl.ds(i * L, L)] = zeros
      return 0
    lax.fori_loop(0, n_own // L, zero_body, 0)

    def count_group(idx, lane_mask):
      cnt, last = plsc.scan_count(idx, lane_mask)
      m = last if lane_mask is None else (last & lane_mask)
      plsc.addupdate_scatter(acc, [idx], cnt.astype(jnp.float32), mask=m)

    def scan_body(i, _):
      count_group(dstv[pl.ds(i * L, L)], None)
      return 0
    lax.fori_loop(0, n_full, scan_body, 0)
    if rem:
      lanes = lax.iota(jnp.int32, L)
      count_group(dstv[pl.ds(n_full * L, L)], lanes < rem)

    pltpu.sync_copy(acc, out_hbm.at[wid])

  return degree_kernel


def _make_degree_reduce_kernel(n_own: int, ro: int):
  mesh = plsc.VectorSubcoreMesh(core_axis_name="c", subcore_axis_name="s")

  @functools.partial(
      pl.kernel,
      out_type=jax.ShapeDtypeStruct((n_own,), jnp.float32),
      mesh=mesh,
      scratch_types=[
          pltpu.VMEM((ro,), jnp.float32),
          pltpu.VMEM((ro,), jnp.float32),
      ],
      compiler_params=pltpu.CompilerParams(needs_layout_passes=False, use_tc_tiling_on_sc=False),
  )
  def degree_reduce_kernel(partial_hbm, out_hbm, acc, tmp):
    cid = lax.axis_index("c")
    sid = lax.axis_index("s")
    wid = sid * NC + cid
    lo = wid * ro

    zeros = jnp.zeros((L,), jnp.float32)
    for g in range(ro // L):
      acc[pl.ds(g * L, L)] = zeros
    for t in range(NW):
      pltpu.sync_copy(partial_hbm.at[t, pl.ds(lo, ro)], tmp)
      for g in range(ro // L):
        plsc.addupdate(acc.at[pl.ds(g * L, L)], tmp[pl.ds(g * L, L)])
    pltpu.sync_copy(acc, out_hbm.at[pl.ds(lo, ro)])

  return degree_reduce_kernel


def _make_agg_kernel(n_nodes: int, e_pad: int, d: int, ro: int, eb: int,
                     gb: int):
  assert e_pad % eb == 0 and eb % L == 0 and d % L == 0 and gb % L == 0
  n_blocks = e_pad // eb
  assert n_blocks % 2 == 0
  n_grp = eb // L
  n_pad = ro * NW
  dl = d // L

  mesh = plsc.VectorSubcoreMesh(core_axis_name="c", subcore_axis_name="s")

  @functools.partial(
      pl.kernel,
      out_type=jax.ShapeDtypeStruct((n_pad, d), jnp.float32),
      mesh=mesh,
      scratch_types=[
          pltpu.VMEM((2, eb), jnp.int32),
          pltpu.VMEM((2, eb), jnp.int32),
          pltpu.VMEM((eb + 2 * gb,), jnp.int32),
          pltpu.VMEM((eb + 2 * gb,), jnp.int32),
          pltpu.VMEM((2, gb, d), jnp.float32),
          pltpu.VMEM((ro + 1, d), jnp.float32),
          pltpu.SemaphoreType.DMA,
          pltpu.SemaphoreType.DMA,
      ],
      compiler_params=pltpu.CompilerParams(needs_layout_passes=False, use_tc_tiling_on_sc=False),
  )
  def agg_kernel(g_hbm, src_hbm, dst_hbm, out_hbm,
                 srcb, dstb, srcf, dstf, rows, acc, sem_e, sem_g):
    cid = lax.axis_index("c")
    sid = lax.axis_index("s")
    wid = sid * NC + cid
    lo = wid * ro

    zeros = jnp.zeros((L,), jnp.float32)
    def zbody(i, _):
      acc[i // dl, pl.ds((i % dl) * L, L)] = zeros
      return 0
    lax.fori_loop(0, (ro + 1) * dl, zbody, 0)

    pad_src = jnp.zeros((L,), jnp.int32)
    pad_dst = jnp.full((L,), ro, jnp.int32)

    def start_edges(bk, par):
      pltpu.async_copy(src_hbm.at[pl.ds(bk * eb, eb)], srcb.at[par], sem_e)
      pltpu.async_copy(dst_hbm.at[pl.ds(bk * eb, eb)], dstb.at[par], sem_e)

    def wait_edges(par):
      pltpu.make_async_copy(src_hbm.at[pl.ds(0, eb)], srcb.at[par],
                            sem_e).wait()
      pltpu.make_async_copy(dst_hbm.at[pl.ds(0, eb)], dstb.at[par],
                            sem_e).wait()

    def start_gather(j, par):
      pltpu.async_copy(g_hbm.at[srcf.at[pl.ds(j * gb, gb)]], rows.at[par],
                       sem_g)

    def wait_gather(par):
      pltpu.make_async_copy(g_hbm.at[srcf.at[pl.ds(0, gb)]], rows.at[par],
                            sem_g).wait()

    def accum(par, j):
      for t in range(gb // L):
        dv = dstf[pl.ds(j * gb + t * L, L)]
        for l in range(L):
          r = dv[l]
          vals = [rows[par, t * L + l, pl.ds(k * L, L)] for k in range(dl)]
          for k in range(dl):
            plsc.addupdate(acc.at[r, pl.ds(k * L, L)], vals[k])

    def process_block(bk, par):
      wait_edges(par)
      @pl.when(bk + 1 < n_blocks)
      def _():
        start_edges(bk + 1, 1 - par)

      def scan_body(i, off):
        d16 = dstb[par, pl.ds(i * L, L)]
        s16 = srcb[par, pl.ds(i * L, L)]
        doff = d16 - lo
        m = doff.astype(jnp.uint32) < jnp.uint32(ro)
        plsc.store_compressed(srcf.at[pl.ds(off, L)], s16, mask=m)
        plsc.store_compressed(dstf.at[pl.ds(off, L)], doff, mask=m)
        return off + plsc.all_reduce_population_count(m)[0]
      off = lax.fori_loop(0, n_grp, scan_body, jnp.int32(0))

      for t in range(gb // L + 1):
        srcf[pl.ds(off + t * L, L)] = pad_src
        dstf[pl.ds(off + t * L, L)] = pad_dst
      nb = (off + gb - 1) // gb

      @pl.when(nb > 0)
      def _():
        start_gather(0, 0)
      def pair_body(j2, _):
        j = 2 * j2
        wait_gather(0)
        @pl.when(j + 1 < nb)
        def _():
          start_gather(j + 1, 1)
        accum(0, j)
        @pl.when(j + 1 < nb)
        def _():
          wait_gather(1)
          @pl.when(j + 2 < nb)
          def _():
            start_gather(j + 2, 0)
          accum(1, j + 1)
        return 0
      lax.fori_loop(0, (nb + 1) // 2, pair_body, 0)

    start_edges(0, 0)
    def block_pair(b2, _):
      process_block(2 * b2, 0)
      process_block(2 * b2 + 1, 1)
      return 0
    lax.fori_loop(0, n_blocks // 2, block_pair, 0)

    pltpu.sync_copy(acc.at[pl.ds(0, ro)], out_hbm.at[pl.ds(lo, ro)])

  return agg_kernel


def _tc_first(deg_ref, x_ref, w_ref, g_ref, dinv_ref):
  dinv = lax.rsqrt(1.0 + deg_ref[...])
  dinv_ref[...] = dinv
  g_ref[...] = jnp.dot(x_ref[...] * dinv, w_ref[...],
                       preferred_element_type=jnp.float32)


def _tc_mid(agg1_ref, agg2_ref, g_ref, dinv_ref, b_ref, w_ref, out_ref):
  t = dinv_ref[...]
  agg = jnp.concatenate([agg1_ref[...], agg2_ref[...]], axis=1)
  u = jnp.maximum(t * (agg + g_ref[...]) + b_ref[...], 0.0)
  out_ref[...] = jnp.dot(u * t, w_ref[...],
                         preferred_element_type=jnp.float32)


def _tc_last(agg_ref, g_ref, dinv_ref, b_ref, out_ref):
  out_ref[...] = dinv_ref[...] * (agg_ref[...] + g_ref[...]) + b_ref[...]


def _row_spec(bm, cols):
  return pl.BlockSpec((bm, cols), lambda i: (i, 0))


def _full_spec(r, c):
  return pl.BlockSpec((r, c), lambda i: (0, 0))


def kernel(x, W1, b1, W2, b2, edge_index):
  n, d_in = x.shape
  d_hid = W1.shape[1]
  d_out = W2.shape[1]
  e = edge_index.shape[1]
  src = edge_index[0]
  dst = edge_index[1]

  bm = 1000
  grid = (n // bm,)

  ro = 320
  n_own = ro * NW
  degp = _make_degree_kernel(n_own, e)(dst)
  deg = _make_degree_reduce_kernel(n_own, ro)(degp)[:n, None]

  g1, dinv = pl.pallas_call(
      _tc_first,
      grid=grid,
      in_specs=[_row_spec(bm, 1),
                _row_spec(bm, d_in), _full_spec(d_in, d_hid)],
      out_specs=[_row_spec(bm, d_hid), _row_spec(bm, 1)],
      out_shape=[jax.ShapeDtypeStruct((n, d_hid), jnp.float32),
                 jax.ShapeDtypeStruct((n, 1), jnp.float32)],
  )(deg, x, W1)

  eb, gb = 4096, 32
  e_pad = ((e + eb - 1) // eb) * eb
  src_p = jnp.concatenate([src, jnp.zeros((e_pad - e,), jnp.int32)])
  dst_p = jnp.concatenate([dst, jnp.full((e_pad - e,), n_own, jnp.int32)])

  dh = d_hid // 2
  agg_h = _make_agg_kernel(n, e_pad, dh, ro=ro, eb=eb, gb=gb)
  agg1a = agg_h(g1[:, :dh], src_p, dst_p)[:n]
  agg1b = agg_h(g1[:, dh:], src_p, dst_p)[:n]

  g2 = pl.pallas_call(
      _tc_mid,
      grid=grid,
      in_specs=[_row_spec(bm, dh), _row_spec(bm, dh), _row_spec(bm, d_hid),
                _row_spec(bm, 1), _full_spec(1, d_hid),
                _full_spec(d_hid, d_out)],
      out_specs=_row_spec(bm, d_out),
      out_shape=jax.ShapeDtypeStruct((n, d_out), jnp.float32),
  )(agg1a, agg1b, g1, dinv, b1[None, :], W2)

  agg2 = _make_agg_kernel(n, e_pad, d_out, ro=ro, eb=eb, gb=gb)(
      g2, src_p, dst_p)[:n]

  out = pl.pallas_call(
      _tc_last,
      grid=grid,
      in_specs=[_row_spec(bm, d_out), _row_spec(bm, d_out), _row_spec(bm, 1),
                _full_spec(1, d_out)],
      out_specs=_row_spec(bm, d_out),
      out_shape=jax.ShapeDtypeStruct((n, d_out), jnp.float32),
  )(agg2, g2, dinv, b2[None, :])

  return out


# speedup vs baseline: 3.6572x; 1.4921x over previous
import functools

import jax
import jax.numpy as jnp
from jax import lax
from jax.experimental import pallas as pl
from jax.experimental.pallas import tpu as pltpu
from jax.experimental.pallas import tpu_sc as plsc

L = 16
NC = 2
NS = 16
NW = NC * NS


def _make_degree_kernel(n_own: int, n_edges: int):
  ept = n_edges // NW
  n_full = ept // L
  rem = ept - n_full * L
  stage = ept + (L - rem) % L

  mesh = plsc.VectorSubcoreMesh(core_axis_name="c", subcore_axis_name="s")

  @functools.partial(
      pl.kernel,
      out_type=jax.ShapeDtypeStruct((NW, n_own), jnp.float32),
      mesh=mesh,
      scratch_types=[
          pltpu.VMEM((stage,), jnp.int32),
          pltpu.VMEM((n_own,), jnp.float32),
      ],
      compiler_params=pltpu.CompilerParams(needs_layout_passes=False, use_tc_tiling_on_sc=False),
  )
  def degree_kernel(dst_hbm, out_hbm, dstv, acc):
    cid = lax.axis_index("c")
    sid = lax.axis_index("s")
    wid = sid * NC + cid
    base = wid * ept

    pltpu.sync_copy(dst_hbm.at[pl.ds(base, ept)], dstv.at[pl.ds(0, ept)])

    zeros = jnp.zeros((L,), jnp.float32)
    def zero_body(i, _):
      acc[pl.ds(i * L, L)] = zeros
      return 0
    lax.fori_loop(0, n_own // L, zero_body, 0)

    def count_group(idx, lane_mask):
      cnt, last = plsc.scan_count(idx, lane_mask)
      m = last if lane_mask is None else (last & lane_mask)
      plsc.addupdate_scatter(acc, [idx], cnt.astype(jnp.float32), mask=m)

    def scan_body(i, _):
      count_group(dstv[pl.ds(i * L, L)], None)
      return 0
    lax.fori_loop(0, n_full, scan_body, 0)
    if rem:
      lanes = lax.iota(jnp.int32, L)
      count_group(dstv[pl.ds(n_full * L, L)], lanes < rem)

    pltpu.sync_copy(acc, out_hbm.at[wid])

  return degree_kernel


def _make_degree_reduce_kernel(n_own: int, ro: int):
  mesh = plsc.VectorSubcoreMesh(core_axis_name="c", subcore_axis_name="s")

  @functools.partial(
      pl.kernel,
      out_type=jax.ShapeDtypeStruct((n_own,), jnp.float32),
      mesh=mesh,
      scratch_types=[
          pltpu.VMEM((ro,), jnp.float32),
          pltpu.VMEM((ro,), jnp.float32),
      ],
      compiler_params=pltpu.CompilerParams(needs_layout_passes=False, use_tc_tiling_on_sc=False),
  )
  def degree_reduce_kernel(partial_hbm, out_hbm, acc, tmp):
    cid = lax.axis_index("c")
    sid = lax.axis_index("s")
    wid = sid * NC + cid
    lo = wid * ro

    zeros = jnp.zeros((L,), jnp.float32)
    for g in range(ro // L):
      acc[pl.ds(g * L, L)] = zeros
    for t in range(NW):
      pltpu.sync_copy(partial_hbm.at[t, pl.ds(lo, ro)], tmp)
      for g in range(ro // L):
        plsc.addupdate(acc.at[pl.ds(g * L, L)], tmp[pl.ds(g * L, L)])
    pltpu.sync_copy(acc, out_hbm.at[pl.ds(lo, ro)])

  return degree_reduce_kernel


def _make_agg_kernel(n_nodes: int, e_pad: int, d: int, ro: int, eb: int,
                     gb: int):
  assert e_pad % eb == 0 and eb % L == 0 and d % L == 0 and gb % L == 0
  n_blocks = e_pad // eb
  assert n_blocks % 2 == 0
  n_grp = eb // L
  n_pad = ro * NW
  dl = d // L

  mesh = plsc.VectorSubcoreMesh(core_axis_name="c", subcore_axis_name="s")

  @functools.partial(
      pl.kernel,
      out_type=jax.ShapeDtypeStruct((n_pad, d), jnp.float32),
      mesh=mesh,
      scratch_types=[
          pltpu.VMEM((2, eb), jnp.int32),
          pltpu.VMEM((2, eb), jnp.int32),
          pltpu.VMEM((eb + 2 * gb,), jnp.int32),
          pltpu.VMEM((eb + 2 * gb,), jnp.int32),
          pltpu.VMEM((2, gb, d), jnp.float32),
          pltpu.VMEM((ro + 1, d), jnp.float32),
          pltpu.SemaphoreType.DMA,
          pltpu.SemaphoreType.DMA,
      ],
      compiler_params=pltpu.CompilerParams(needs_layout_passes=False, use_tc_tiling_on_sc=False),
  )
  def agg_kernel(g_hbm, src_hbm, dst_hbm, out_hbm,
                 srcb, dstb, srcf, dstf, rows, acc, sem_e, sem_g):
    cid = lax.axis_index("c")
    sid = lax.axis_index("s")
    wid = sid * NC + cid
    lo = wid * ro

    zeros = jnp.zeros((L,), jnp.float32)
    def zbody(i, _):
      acc[i // dl, pl.ds((i % dl) * L, L)] = zeros
      return 0
    lax.fori_loop(0, (ro + 1) * dl, zbody, 0)

    pad_src = wid * 312 + lax.iota(jnp.int32, L)
    pad_dst = jnp.full((L,), ro, jnp.int32)

    def start_edges(bk, par):
      pltpu.async_copy(src_hbm.at[pl.ds(bk * eb, eb)], srcb.at[par], sem_e)
      pltpu.async_copy(dst_hbm.at[pl.ds(bk * eb, eb)], dstb.at[par], sem_e)

    def wait_edges(par):
      pltpu.make_async_copy(src_hbm.at[pl.ds(0, eb)], srcb.at[par],
                            sem_e).wait()
      pltpu.make_async_copy(dst_hbm.at[pl.ds(0, eb)], dstb.at[par],
                            sem_e).wait()

    def start_gather(j, par):
      pltpu.async_copy(g_hbm.at[srcf.at[pl.ds(j * gb, gb)]], rows.at[par],
                       sem_g)

    def wait_gather(par):
      pltpu.make_async_copy(g_hbm.at[srcf.at[pl.ds(0, gb)]], rows.at[par],
                            sem_g).wait()

    def accum(par, j):
      for t in range(gb // L):
        dv = dstf[pl.ds(j * gb + t * L, L)]
        for l in range(L):
          r = dv[l]
          vals = [rows[par, t * L + l, pl.ds(k * L, L)] for k in range(dl)]
          for k in range(dl):
            plsc.addupdate(acc.at[r, pl.ds(k * L, L)], vals[k])

    def process_block(bk, par):
      wait_edges(par)
      @pl.when(bk + 1 < n_blocks)
      def _():
        start_edges(bk + 1, 1 - par)

      def scan_body(i, off):
        d16 = dstb[par, pl.ds(i * L, L)]
        s16 = srcb[par, pl.ds(i * L, L)]
        doff = d16 - lo
        m = doff.astype(jnp.uint32) < jnp.uint32(ro)
        plsc.store_compressed(srcf.at[pl.ds(off, L)], s16, mask=m)
        plsc.store_compressed(dstf.at[pl.ds(off, L)], doff, mask=m)
        return off + plsc.all_reduce_population_count(m)[0]
      off = lax.fori_loop(0, n_grp, scan_body, jnp.int32(0))

      for t in range(gb // L + 1):
        srcf[pl.ds(off + t * L, L)] = pad_src
        dstf[pl.ds(off + t * L, L)] = pad_dst
      nb = (off + gb - 1) // gb

      @pl.when(nb > 0)
      def _():
        start_gather(0, 0)
      def pair_body(j2, _):
        j = 2 * j2
        wait_gather(0)
        @pl.when(j + 1 < nb)
        def _():
          start_gather(j + 1, 1)
        accum(0, j)
        @pl.when(j + 1 < nb)
        def _():
          wait_gather(1)
          @pl.when(j + 2 < nb)
          def _():
            start_gather(j + 2, 0)
          accum(1, j + 1)
        return 0
      lax.fori_loop(0, (nb + 1) // 2, pair_body, 0)

    start_edges(0, 0)
    def block_pair(b2, _):
      process_block(2 * b2, 0)
      process_block(2 * b2 + 1, 1)
      return 0
    lax.fori_loop(0, n_blocks // 2, block_pair, 0)

    pltpu.sync_copy(acc.at[pl.ds(0, ro)], out_hbm.at[pl.ds(lo, ro)])

  return agg_kernel


def _tc_first(deg_ref, x_ref, w_ref, g_ref, dinv_ref):
  dinv = lax.rsqrt(1.0 + deg_ref[...])
  dinv_ref[...] = dinv
  g_ref[...] = jnp.dot(x_ref[...] * dinv, w_ref[...],
                       preferred_element_type=jnp.float32)


def _tc_mid(agg1_ref, agg2_ref, g_ref, dinv_ref, b_ref, w_ref, out_ref):
  t = dinv_ref[...]
  agg = jnp.concatenate([agg1_ref[...], agg2_ref[...]], axis=1)
  u = jnp.maximum(t * (agg + g_ref[...]) + b_ref[...], 0.0)
  out_ref[...] = jnp.dot(u * t, w_ref[...],
                         preferred_element_type=jnp.float32)


def _tc_last(agg_ref, g_ref, dinv_ref, b_ref, out_ref):
  out_ref[...] = dinv_ref[...] * (agg_ref[...] + g_ref[...]) + b_ref[...]


def _row_spec(bm, cols):
  return pl.BlockSpec((bm, cols), lambda i: (i, 0))


def _full_spec(r, c):
  return pl.BlockSpec((r, c), lambda i: (0, 0))


def kernel(x, W1, b1, W2, b2, edge_index):
  n, d_in = x.shape
  d_hid = W1.shape[1]
  d_out = W2.shape[1]
  e = edge_index.shape[1]
  src = edge_index[0]
  dst = edge_index[1]

  bm = 1000
  grid = (n // bm,)

  ro = 320
  n_own = ro * NW
  degp = _make_degree_kernel(n_own, e)(dst)
  deg = _make_degree_reduce_kernel(n_own, ro)(degp)[:n, None]

  g1, dinv = pl.pallas_call(
      _tc_first,
      grid=grid,
      in_specs=[_row_spec(bm, 1),
                _row_spec(bm, d_in), _full_spec(d_in, d_hid)],
      out_specs=[_row_spec(bm, d_hid), _row_spec(bm, 1)],
      out_shape=[jax.ShapeDtypeStruct((n, d_hid), jnp.float32),
                 jax.ShapeDtypeStruct((n, 1), jnp.float32)],
  )(deg, x, W1)

  eb, gb = 4096, 32
  e_pad = ((e + eb - 1) // eb) * eb
  src_p = jnp.concatenate([src, jnp.zeros((e_pad - e,), jnp.int32)])
  dst_p = jnp.concatenate([dst, jnp.full((e_pad - e,), n_own, jnp.int32)])

  dh = d_hid // 2
  agg_h = _make_agg_kernel(n, e_pad, dh, ro=ro, eb=eb, gb=gb)
  agg1a = agg_h(g1[:, :dh], src_p, dst_p)[:n]
  agg1b = agg_h(g1[:, dh:], src_p, dst_p)[:n]

  g2 = pl.pallas_call(
      _tc_mid,
      grid=grid,
      in_specs=[_row_spec(bm, dh), _row_spec(bm, dh), _row_spec(bm, d_hid),
                _row_spec(bm, 1), _full_spec(1, d_hid),
                _full_spec(d_hid, d_out)],
      out_specs=_row_spec(bm, d_out),
      out_shape=jax.ShapeDtypeStruct((n, d_out), jnp.float32),
  )(agg1a, agg1b, g1, dinv, b1[None, :], W2)

  agg2 = _make_agg_kernel(n, e_pad, d_out, ro=ro, eb=eb, gb=gb)(
      g2, src_p, dst_p)[:n]

  out = pl.pallas_call(
      _tc_last,
      grid=grid,
      in_specs=[_row_spec(bm, d_out), _row_spec(bm, d_out), _row_spec(bm, 1),
                _full_spec(1, d_out)],
      out_specs=_row_spec(bm, d_out),
      out_shape=jax.ShapeDtypeStruct((n, d_out), jnp.float32),
  )(agg2, g2, dinv, b2[None, :])

  return out


# speedup vs baseline: 3.6995x; 1.0116x over previous
import functools

import jax
import jax.numpy as jnp
from jax import lax
from jax.experimental import pallas as pl
from jax.experimental.pallas import tpu as pltpu
from jax.experimental.pallas import tpu_sc as plsc

L = 16
NC = 2
NS = 16
NW = NC * NS


def _make_degree_kernel(n_own: int, n_edges: int):
  ept = n_edges // NW
  n_full = ept // L
  rem = ept - n_full * L
  stage = ept + (L - rem) % L

  mesh = plsc.VectorSubcoreMesh(core_axis_name="c", subcore_axis_name="s")

  @functools.partial(
      pl.kernel,
      out_type=jax.ShapeDtypeStruct((NW, n_own), jnp.float32),
      mesh=mesh,
      scratch_types=[
          pltpu.VMEM((stage,), jnp.int32),
          pltpu.VMEM((n_own,), jnp.float32),
      ],
      compiler_params=pltpu.CompilerParams(needs_layout_passes=False, use_tc_tiling_on_sc=False),
  )
  def degree_kernel(dst_hbm, out_hbm, dstv, acc):
    cid = lax.axis_index("c")
    sid = lax.axis_index("s")
    wid = sid * NC + cid
    base = wid * ept

    pltpu.sync_copy(dst_hbm.at[pl.ds(base, ept)], dstv.at[pl.ds(0, ept)])

    zeros = jnp.zeros((L,), jnp.float32)
    def zero_body(i, _):
      acc[pl.ds(i * L, L)] = zeros
      return 0
    lax.fori_loop(0, n_own // L, zero_body, 0)

    def count_group(idx, lane_mask):
      cnt, last = plsc.scan_count(idx, lane_mask)
      m = last if lane_mask is None else (last & lane_mask)
      plsc.addupdate_scatter(acc, [idx], cnt.astype(jnp.float32), mask=m)

    def scan_body(i, _):
      count_group(dstv[pl.ds(i * L, L)], None)
      return 0
    lax.fori_loop(0, n_full, scan_body, 0)
    if rem:
      lanes = lax.iota(jnp.int32, L)
      count_group(dstv[pl.ds(n_full * L, L)], lanes < rem)

    pltpu.sync_copy(acc, out_hbm.at[wid])

  return degree_kernel


def _make_degree_reduce_kernel(n_own: int, ro: int):
  mesh = plsc.VectorSubcoreMesh(core_axis_name="c", subcore_axis_name="s")

  @functools.partial(
      pl.kernel,
      out_type=jax.ShapeDtypeStruct((n_own,), jnp.float32),
      mesh=mesh,
      scratch_types=[
          pltpu.VMEM((ro,), jnp.float32),
          pltpu.VMEM((ro,), jnp.float32),
      ],
      compiler_params=pltpu.CompilerParams(needs_layout_passes=False, use_tc_tiling_on_sc=False),
  )
  def degree_reduce_kernel(partial_hbm, out_hbm, acc, tmp):
    cid = lax.axis_index("c")
    sid = lax.axis_index("s")
    wid = sid * NC + cid
    lo = wid * ro

    zeros = jnp.zeros((L,), jnp.float32)
    for g in range(ro // L):
      acc[pl.ds(g * L, L)] = zeros
    for t in range(NW):
      pltpu.sync_copy(partial_hbm.at[t, pl.ds(lo, ro)], tmp)
      for g in range(ro // L):
        plsc.addupdate(acc.at[pl.ds(g * L, L)], tmp[pl.ds(g * L, L)])
    pltpu.sync_copy(acc, out_hbm.at[pl.ds(lo, ro)])

  return degree_reduce_kernel


def _make_agg_kernel(n_nodes: int, e_pad: int, d: int, ro: int, eb: int,
                     gb: int):
  assert e_pad % eb == 0 and eb % L == 0 and d % L == 0 and gb % L == 0
  n_blocks = e_pad // eb
  assert n_blocks % 2 == 0
  n_grp = eb // L
  n_pad = ro * NW
  dl = d // L

  mesh = plsc.VectorSubcoreMesh(core_axis_name="c", subcore_axis_name="s")

  @functools.partial(
      pl.kernel,
      out_type=jax.ShapeDtypeStruct((n_pad, d), jnp.float32),
      mesh=mesh,
      scratch_types=[
          pltpu.VMEM((2, eb), jnp.int32),
          pltpu.VMEM((2, eb), jnp.int32),
          pltpu.VMEM((eb + 2 * gb,), jnp.int32),
          pltpu.VMEM((eb + 2 * gb,), jnp.int32),
          pltpu.VMEM((2, gb, d), jnp.float32),
          pltpu.VMEM((ro + 1, d), jnp.float32),
          pltpu.SemaphoreType.DMA,
          pltpu.SemaphoreType.DMA,
      ],
      compiler_params=pltpu.CompilerParams(needs_layout_passes=False, use_tc_tiling_on_sc=False),
  )
  def agg_kernel(g_hbm, src_hbm, dst_hbm, out_hbm,
                 srcb, dstb, srcf, dstf, rows, acc, sem_e, sem_g):
    cid = lax.axis_index("c")
    sid = lax.axis_index("s")
    wid = sid * NC + cid
    lo = wid * ro

    zeros = jnp.zeros((L,), jnp.float32)
    def zbody(i, _):
      acc[i // dl, pl.ds((i % dl) * L, L)] = zeros
      return 0
    lax.fori_loop(0, (ro + 1) * dl, zbody, 0)

    pad_src = wid * 312 + lax.iota(jnp.int32, L)
    pad_dst = jnp.full((L,), ro, jnp.int32)

    def start_edges(bk, par):
      pltpu.async_copy(src_hbm.at[pl.ds(bk * eb, eb)], srcb.at[par], sem_e)
      pltpu.async_copy(dst_hbm.at[pl.ds(bk * eb, eb)], dstb.at[par], sem_e)

    def wait_edges(par):
      pltpu.make_async_copy(src_hbm.at[pl.ds(0, eb)], srcb.at[par],
                            sem_e).wait()
      pltpu.make_async_copy(dst_hbm.at[pl.ds(0, eb)], dstb.at[par],
                            sem_e).wait()

    def start_gather(j, par):
      pltpu.async_copy(g_hbm.at[srcf.at[pl.ds(j * gb, gb)]], rows.at[par],
                       sem_g)

    def wait_gather(par):
      pltpu.make_async_copy(g_hbm.at[srcf.at[pl.ds(0, gb)]], rows.at[par],
                            sem_g).wait()

    def accum(par, j):
      prev_r = prev_vals = None
      for t in range(gb // L):
        dv = dstf[pl.ds(j * gb + t * L, L)]
        for l in range(L):
          r = dv[l]
          cur = []
          for k in range(dl):
            cur.append(rows[par, t * L + l, pl.ds(k * L, L)])
            if prev_vals is not None:
              plsc.addupdate(acc.at[prev_r, pl.ds(k * L, L)], prev_vals[k])
          prev_r, prev_vals = r, cur
      for k in range(dl):
        plsc.addupdate(acc.at[prev_r, pl.ds(k * L, L)], prev_vals[k])

    def process_block(bk, par):
      wait_edges(par)
      @pl.when(bk + 1 < n_blocks)
      def _():
        start_edges(bk + 1, 1 - par)

      def scan_one(i, off):
        d16 = dstb[par, pl.ds(i * L, L)]
        s16 = srcb[par, pl.ds(i * L, L)]
        doff = d16 - lo
        m = doff.astype(jnp.uint32) < jnp.uint32(ro)
        plsc.store_compressed(srcf.at[pl.ds(off, L)], s16, mask=m)
        plsc.store_compressed(dstf.at[pl.ds(off, L)], doff, mask=m)
        return off + plsc.all_reduce_population_count(m)[0]
      def scan_body(i2, off):
        return scan_one(2 * i2 + 1, scan_one(2 * i2, off))
      off = lax.fori_loop(0, n_grp // 2, scan_body, jnp.int32(0))

      for t in range(gb // L + 1):
        srcf[pl.ds(off + t * L, L)] = pad_src
        dstf[pl.ds(off + t * L, L)] = pad_dst
      nb = (off + gb - 1) // gb

      @pl.when(nb > 0)
      def _():
        start_gather(0, 0)
      def pair_body(j2, _):
        j = 2 * j2
        wait_gather(0)
        @pl.when(j + 1 < nb)
        def _():
          start_gather(j + 1, 1)
        accum(0, j)
        @pl.when(j + 1 < nb)
        def _():
          wait_gather(1)
          @pl.when(j + 2 < nb)
          def _():
            start_gather(j + 2, 0)
          accum(1, j + 1)
        return 0
      lax.fori_loop(0, (nb + 1) // 2, pair_body, 0)

    start_edges(0, 0)
    def block_pair(b2, _):
      process_block(2 * b2, 0)
      process_block(2 * b2 + 1, 1)
      return 0
    lax.fori_loop(0, n_blocks // 2, block_pair, 0)

    pltpu.sync_copy(acc.at[pl.ds(0, ro)], out_hbm.at[pl.ds(lo, ro)])

  return agg_kernel


def _tc_first(deg_ref, x_ref, w_ref, g_ref, dinv_ref):
  dinv = lax.rsqrt(1.0 + deg_ref[...])
  dinv_ref[...] = dinv
  g_ref[...] = jnp.dot(x_ref[...] * dinv, w_ref[...],
                       preferred_element_type=jnp.float32)


def _tc_mid(agg1_ref, agg2_ref, g_ref, dinv_ref, b_ref, w_ref, out_ref):
  t = dinv_ref[...]
  agg = jnp.concatenate([agg1_ref[...], agg2_ref[...]], axis=1)
  u = jnp.maximum(t * (agg + g_ref[...]) + b_ref[...], 0.0)
  out_ref[...] = jnp.dot(u * t, w_ref[...],
                         preferred_element_type=jnp.float32)


def _tc_last(agg_ref, g_ref, dinv_ref, b_ref, out_ref):
  out_ref[...] = dinv_ref[...] * (agg_ref[...] + g_ref[...]) + b_ref[...]


def _row_spec(bm, cols):
  return pl.BlockSpec((bm, cols), lambda i: (i, 0))


def _full_spec(r, c):
  return pl.BlockSpec((r, c), lambda i: (0, 0))


def kernel(x, W1, b1, W2, b2, edge_index):
  n, d_in = x.shape
  d_hid = W1.shape[1]
  d_out = W2.shape[1]
  e = edge_index.shape[1]
  src = edge_index[0]
  dst = edge_index[1]

  bm = 1000
  grid = (n // bm,)

  ro = 320
  n_own = ro * NW
  degp = _make_degree_kernel(n_own, e)(dst)
  deg = _make_degree_reduce_kernel(n_own, ro)(degp)[:n, None]

  g1, dinv = pl.pallas_call(
      _tc_first,
      grid=grid,
      in_specs=[_row_spec(bm, 1),
                _row_spec(bm, d_in), _full_spec(d_in, d_hid)],
      out_specs=[_row_spec(bm, d_hid), _row_spec(bm, 1)],
      out_shape=[jax.ShapeDtypeStruct((n, d_hid), jnp.float32),
                 jax.ShapeDtypeStruct((n, 1), jnp.float32)],
  )(deg, x, W1)

  eb, gb = 4096, 32
  e_pad = ((e + eb - 1) // eb) * eb
  src_p = jnp.concatenate([src, jnp.zeros((e_pad - e,), jnp.int32)])
  dst_p = jnp.concatenate([dst, jnp.full((e_pad - e,), n_own, jnp.int32)])

  dh = d_hid // 2
  agg_h = _make_agg_kernel(n, e_pad, dh, ro=ro, eb=eb, gb=gb)
  agg1a = agg_h(g1[:, :dh], src_p, dst_p)[:n]
  agg1b = agg_h(g1[:, dh:], src_p, dst_p)[:n]

  g2 = pl.pallas_call(
      _tc_mid,
      grid=grid,
      in_specs=[_row_spec(bm, dh), _row_spec(bm, dh), _row_spec(bm, d_hid),
                _row_spec(bm, 1), _full_spec(1, d_hid),
                _full_spec(d_hid, d_out)],
      out_specs=_row_spec(bm, d_out),
      out_shape=jax.ShapeDtypeStruct((n, d_out), jnp.float32),
  )(agg1a, agg1b, g1, dinv, b1[None, :], W2)

  agg2 = _make_agg_kernel(n, e_pad, d_out, ro=ro, eb=eb, gb=gb)(
      g2, src_p, dst_p)[:n]

  out = pl.pallas_call(
      _tc_last,
      grid=grid,
      in_specs=[_row_spec(bm, d_out), _row_spec(bm, d_out), _row_spec(bm, 1),
                _full_spec(1, d_out)],
      out_specs=_row_spec(bm, d_out),
      out_shape=jax.ShapeDtypeStruct((n, d_out), jnp.float32),
  )(agg2, g2, dinv, b2[None, :])

  return out
